# Initial kernel scaffold; baseline (speedup 1.0000x reference)
import jax
import jax.numpy as jnp
from jax.experimental import pallas as pl


def kernel(x33, x26, gamma, beta, conv_w):
    raise NotImplementedError("write your pallas kernel here")


# speedup vs baseline: 1.1694x; 1.1694x over previous
import functools

import jax
import jax.numpy as jnp
from jax.experimental import pallas as pl
from jax.experimental.pallas import tpu as pltpu


def _lane_tile(tn, hw):
    if tn >= hw:
        return hw
    return max(128, (tn // 128) * 128)


def _stats_kernel(x_ref, sum_ref, sq_ref):
    x = x_ref[...]
    sum_ref[...] = jnp.sum(x, axis=-1, keepdims=True)
    sq_ref[...] = jnp.sum(x * x, axis=-1, keepdims=True)


def _fused_kernel(x_ref, r_ref, psum_ref, psq_ref, gamma_ref, beta_ref,
                  w_ref, o_ref, *, count, eps):
    total = jnp.sum(psum_ref[...], axis=0)
    total_sq = jnp.sum(psq_ref[...], axis=0)
    inv_count = 1.0 / count
    mean = total * inv_count
    var = total_sq * inv_count - mean * mean
    inv_std = jax.lax.rsqrt(var + eps)
    scale = gamma_ref[...] * inv_std
    shift = beta_ref[...] - mean * scale

    y = jnp.maximum(x_ref[...] * scale + shift + r_ref[...], 0.0)
    o_ref[...] = jnp.dot(w_ref[...], y.astype(jnp.bfloat16),
                         preferred_element_type=jnp.float32)


@functools.partial(jax.jit, static_argnames=("tn",))
def _forward(x33, x26, gamma, beta, conv_w, *, tn=512):
    N, Cin, H, W = x33.shape
    Cout = conv_w.shape[0]
    HW = H * W

    x = x33.reshape(N, Cin, HW).astype(jnp.float32)
    r = x26.reshape(N, Cin, HW).astype(jnp.float32)

    psum, psq = pl.pallas_call(
        _stats_kernel,
        out_shape=(
            jax.ShapeDtypeStruct((N, Cin, 1), jnp.float32),
            jax.ShapeDtypeStruct((N, Cin, 1), jnp.float32),
        ),
        grid=(N,),
        in_specs=[pl.BlockSpec((None, Cin, HW), lambda n: (n, 0, 0))],
        out_specs=(
            pl.BlockSpec((None, Cin, 1), lambda n: (n, 0, 0)),
            pl.BlockSpec((None, Cin, 1), lambda n: (n, 0, 0)),
        ),
        compiler_params=pltpu.CompilerParams(
            dimension_semantics=("parallel",)),
    )(x)

    w = conv_w.reshape(Cout, Cin).astype(jnp.bfloat16)
    g2 = gamma.reshape(Cin, 1).astype(jnp.float32)
    b2 = beta.reshape(Cin, 1).astype(jnp.float32)

    tn = _lane_tile(tn, HW)
    nblk = pl.cdiv(HW, tn)
    out = pl.pallas_call(
        functools.partial(_fused_kernel, count=N * HW, eps=1e-5),
        out_shape=jax.ShapeDtypeStruct((N, Cout, HW), jnp.float32),
        grid=(N, nblk),
        in_specs=[
            pl.BlockSpec((None, Cin, tn), lambda n, j: (n, 0, j)),
            pl.BlockSpec((None, Cin, tn), lambda n, j: (n, 0, j)),
            pl.BlockSpec((N, Cin, 1), lambda n, j: (0, 0, 0)),
            pl.BlockSpec((N, Cin, 1), lambda n, j: (0, 0, 0)),
            pl.BlockSpec((Cin, 1), lambda n, j: (0, 0)),
            pl.BlockSpec((Cin, 1), lambda n, j: (0, 0)),
            pl.BlockSpec((Cout, Cin), lambda n, j: (0, 0)),
        ],
        out_specs=pl.BlockSpec((None, Cout, tn), lambda n, j: (n, 0, j)),
        compiler_params=pltpu.CompilerParams(
            dimension_semantics=("parallel", "parallel")),
    )(x, r, psum, psq, g2, b2, w)
    return out.reshape(N, Cout, H, W)


def kernel(x33, x26, gamma, beta, conv_w):
    return _forward(x33, x26, gamma, beta, conv_w)


# speedup vs baseline: 1.1721x; 1.0024x over previous
import functools

import jax
import jax.numpy as jnp
from jax.experimental import pallas as pl
from jax.experimental.pallas import tpu as pltpu


def _lane_tile(tn, hw):
    if tn >= hw:
        return hw
    return max(128, (tn // 128) * 128)


def _stats_kernel(x_ref, sum_ref, sq_ref):
    x = x_ref[...]
    sum_ref[...] = jnp.sum(x, axis=-1, keepdims=True)
    sq_ref[...] = jnp.sum(x * x, axis=-1, keepdims=True)


def _fused_kernel(x_ref, r_ref, psum_ref, psq_ref, gamma_ref, beta_ref,
                  w_ref, o_ref, *, count, eps):
    total = jnp.sum(psum_ref[...], axis=0)
    total_sq = jnp.sum(psq_ref[...], axis=0)
    inv_count = 1.0 / count
    mean = total * inv_count
    var = total_sq * inv_count - mean * mean
    inv_std = jax.lax.rsqrt(var + eps)
    scale = gamma_ref[...] * inv_std
    shift = beta_ref[...] - mean * scale

    y = jnp.maximum(x_ref[...] * scale + shift + r_ref[...], 0.0)
    o_ref[...] = jnp.dot(w_ref[...].astype(jnp.bfloat16),
                         y.astype(jnp.bfloat16),
                         preferred_element_type=jnp.float32)


@functools.partial(jax.jit, static_argnames=("tn",))
def _forward(x33, x26, gamma, beta, conv_w, *, tn=512):
    N, Cin, H, W = x33.shape
    Cout = conv_w.shape[0]
    HW = H * W

    x = x33.reshape(N, Cin, HW).astype(jnp.float32)
    r = x26.reshape(N, Cin, HW).astype(jnp.float32)

    psum, psq = pl.pallas_call(
        _stats_kernel,
        out_shape=(
            jax.ShapeDtypeStruct((N, Cin, 1), jnp.float32),
            jax.ShapeDtypeStruct((N, Cin, 1), jnp.float32),
        ),
        grid=(N,),
        in_specs=[pl.BlockSpec((None, Cin, HW), lambda n: (n, 0, 0))],
        out_specs=(
            pl.BlockSpec((None, Cin, 1), lambda n: (n, 0, 0)),
            pl.BlockSpec((None, Cin, 1), lambda n: (n, 0, 0)),
        ),
        compiler_params=pltpu.CompilerParams(
            dimension_semantics=("parallel",)),
    )(x)

    w = conv_w.reshape(Cout, Cin)
    g2 = gamma.reshape(Cin, 1)
    b2 = beta.reshape(Cin, 1)

    tn = _lane_tile(tn, HW)
    nblk = pl.cdiv(HW, tn)
    out = pl.pallas_call(
        functools.partial(_fused_kernel, count=N * HW, eps=1e-5),
        out_shape=jax.ShapeDtypeStruct((N, Cout, HW), jnp.float32),
        grid=(N, nblk),
        in_specs=[
            pl.BlockSpec((None, Cin, tn), lambda n, j: (n, 0, j)),
            pl.BlockSpec((None, Cin, tn), lambda n, j: (n, 0, j)),
            pl.BlockSpec((N, Cin, 1), lambda n, j: (0, 0, 0)),
            pl.BlockSpec((N, Cin, 1), lambda n, j: (0, 0, 0)),
            pl.BlockSpec((Cin, 1), lambda n, j: (0, 0)),
            pl.BlockSpec((Cin, 1), lambda n, j: (0, 0)),
            pl.BlockSpec((Cout, Cin), lambda n, j: (0, 0)),
        ],
        out_specs=pl.BlockSpec((None, Cout, tn), lambda n, j: (n, 0, j)),
        compiler_params=pltpu.CompilerParams(
            dimension_semantics=("parallel", "parallel")),
    )(x, r, psum, psq, g2, b2, w)
    return out.reshape(N, Cout, H, W)


def kernel(x33, x26, gamma, beta, conv_w):
    return _forward(x33, x26, gamma, beta, conv_w)
